# baseline (speedup 1.0000x reference)
import jax
import jax.numpy as jnp
from jax.experimental import pallas as pl
from jax.experimental.pallas import tpu as pltpu

BATCH = 16384
ATOMS = 200
BLK = 2048


def _tc_body(se_ref, en_ref, sp_ref, out_ref):
    s0 = se_ref[0]
    s1 = se_ref[1]
    s2 = se_ref[2]
    s3 = se_ref[3]
    sp = sp_ref[...]
    lo = jnp.where(sp == 0, s0, s1)
    hi = jnp.where(sp == 2, s2, s3)
    atomic = jnp.where(sp < 2, lo, hi)
    sae = jnp.sum(atomic, axis=1)
    out_ref[0, 0, :] = en_ref[0, 0, :] - sae


def kernel(energies, species, self_energies):
    nb = BATCH // BLK
    en3 = energies.reshape(nb, 1, BLK)
    out = pl.pallas_call(
        _tc_body,
        grid=(nb,),
        in_specs=[
            pl.BlockSpec(memory_space=pltpu.SMEM),
            pl.BlockSpec((1, 1, BLK), lambda i: (i, 0, 0)),
            pl.BlockSpec((BLK, ATOMS), lambda i: (i, 0)),
        ],
        out_specs=pl.BlockSpec((1, 1, BLK), lambda i: (i, 0, 0)),
        out_shape=jax.ShapeDtypeStruct((nb, 1, BLK), jnp.float32),
    )(self_energies, en3, species)
    return out.reshape(BATCH)
